# Initial kernel scaffold; baseline (speedup 1.0000x reference)
import jax
import jax.numpy as jnp
from jax.experimental import pallas as pl


def kernel(hidden_states, gate_w, Wg, Wu, Wd, Wg_s, Wu_s, Wd_s):
    raise NotImplementedError("write your pallas kernel here")


# speedup vs baseline: 1.3451x; 1.3451x over previous
import functools

import jax
import jax.numpy as jnp
from jax.experimental import pallas as pl
from jax.experimental.pallas import tpu as pltpu

D, F, E, K, NSH = 2048, 5120, 8, 2, 2
BM = 256
BF = 512
NF = F // BF
NB = 23
P = NB * BM

BM2 = 512
T = 2048


def _routed_body(be_ref, x_ref, wg_ref, wu_ref, wd_ref, ws_ref, out_ref):
    b = pl.program_id(0)
    f = pl.program_id(1)

    @pl.when(f == 0)
    def _():
        out_ref[...] = jnp.zeros_like(out_ref)

    @pl.when(be_ref[b] < E)
    def _():
        x = x_ref[...]
        g = jnp.dot(x, wg_ref[0], preferred_element_type=jnp.float32)
        u = jnp.dot(x, wu_ref[0], preferred_element_type=jnp.float32)
        h = (g * jax.nn.sigmoid(g)) * u
        out_ref[...] += jnp.dot(h, wd_ref[0], preferred_element_type=jnp.float32)

    @pl.when(f == NF - 1)
    def _():
        out_ref[...] *= ws_ref[0, 0, :][:, None]


def _routed_mlp(block_expert, xg, Wg, Wu, Wd, wslot):
    grid_spec = pltpu.PrefetchScalarGridSpec(
        num_scalar_prefetch=1,
        grid=(NB, NF),
        in_specs=[
            pl.BlockSpec((BM, D), lambda b, f, be: (b, 0)),
            pl.BlockSpec((1, D, BF), lambda b, f, be: (jnp.minimum(be[b], E - 1), 0, f)),
            pl.BlockSpec((1, D, BF), lambda b, f, be: (jnp.minimum(be[b], E - 1), 0, f)),
            pl.BlockSpec((1, BF, D), lambda b, f, be: (jnp.minimum(be[b], E - 1), f, 0)),
            pl.BlockSpec((1, 1, BM), lambda b, f, be: (b, 0, 0)),
        ],
        out_specs=pl.BlockSpec((BM, D), lambda b, f, be: (b, 0)),
    )
    return pl.pallas_call(
        _routed_body,
        grid_spec=grid_spec,
        out_shape=jax.ShapeDtypeStruct((P, D), jnp.float32),
    )(block_expert, xg, Wg, Wu, Wd, wslot.reshape(NB, 1, BM))


def _shared_body(x_ref, wg_ref, wu_ref, wd_ref, out_ref):
    si = pl.program_id(1)
    f = pl.program_id(2)

    @pl.when((si == 0) & (f == 0))
    def _():
        out_ref[...] = jnp.zeros_like(out_ref)

    x = x_ref[...]
    g = jnp.dot(x, wg_ref[0], preferred_element_type=jnp.float32)
    u = jnp.dot(x, wu_ref[0], preferred_element_type=jnp.float32)
    h = (g * jax.nn.sigmoid(g)) * u
    out_ref[...] += jnp.dot(h, wd_ref[0], preferred_element_type=jnp.float32)


def _shared_mlp(x, Wg_s, Wu_s, Wd_s):
    return pl.pallas_call(
        _shared_body,
        grid=(T // BM2, NSH, NF),
        in_specs=[
            pl.BlockSpec((BM2, D), lambda tb, si, f: (tb, 0)),
            pl.BlockSpec((1, D, BF), lambda tb, si, f: (si, 0, f)),
            pl.BlockSpec((1, D, BF), lambda tb, si, f: (si, 0, f)),
            pl.BlockSpec((1, BF, D), lambda tb, si, f: (si, f, 0)),
        ],
        out_specs=pl.BlockSpec((BM2, D), lambda tb, si, f: (tb, 0)),
        out_shape=jax.ShapeDtypeStruct((T, D), jnp.float32),
    )(x, Wg_s, Wu_s, Wd_s)


def kernel(hidden_states, gate_w, Wg, Wu, Wd, Wg_s, Wu_s, Wd_s):
    b, s, d = hidden_states.shape
    x = hidden_states.reshape(-1, d)

    logits = x @ gate_w.T
    scores = jax.nn.sigmoid(logits)
    topk_w, topk_idx = jax.lax.top_k(scores, K)
    topk_w = topk_w / jnp.sum(topk_w, axis=-1, keepdims=True)

    flat_e = topk_idx.reshape(-1).astype(jnp.int32)
    flat_w = topk_w.reshape(-1)
    order = jnp.argsort(flat_e, stable=True).astype(jnp.int32)
    e_sorted = flat_e[order]
    counts = jnp.bincount(flat_e, length=E).astype(jnp.int32)
    padded = ((counts + BM - 1) // BM) * BM
    cum_padded = jnp.cumsum(padded)
    p_off = cum_padded - padded
    g_off = jnp.cumsum(counts) - counts
    j = jnp.arange(T * K, dtype=jnp.int32)
    dest = p_off[e_sorted] + (j - g_off[e_sorted])
    perm = jnp.zeros((P,), jnp.int32).at[dest].set(order // K)
    wslot = jnp.zeros((P,), jnp.float32).at[dest].set(flat_w[order])
    pos = jnp.zeros((T * K,), jnp.int32).at[order].set(dest).reshape(T, K)
    block_expert = jnp.searchsorted(
        cum_padded, jnp.arange(NB, dtype=jnp.int32) * BM, side="right"
    ).astype(jnp.int32)

    xg = x[perm]
    yg = _routed_mlp(block_expert, xg, Wg, Wu, Wd, wslot)
    out_sh = _shared_mlp(x, Wg_s, Wu_s, Wd_s)
    out = out_sh + yg[pos[:, 0]] + yg[pos[:, 1]]
    return out.reshape(b, s, d)


# speedup vs baseline: 1.3934x; 1.0359x over previous
import functools

import jax
import jax.numpy as jnp
from jax import lax
from jax.experimental import pallas as pl
from jax.experimental.pallas import tpu as pltpu
from jax.experimental.pallas import tpu_sc as plsc

D, F, E, K, NSH = 2048, 5120, 8, 2, 2
T = 2048
BM = 256
BF = 512
NF = F // BF
NB = 23
P = NB * BM
BM2 = 512

_NW = 32
_GCH = 8
_CCH = 16

_MESH = plsc.VectorSubcoreMesh(core_axis_name="c", subcore_axis_name="s")


def _sc_gather(x, perm):
    bpw = P // _NW
    nit = bpw // _GCH

    @functools.partial(
        pl.kernel, mesh=_MESH,
        out_type=jax.ShapeDtypeStruct((P, D), jnp.float32),
        scratch_types=[
            pltpu.VMEM((nit, _GCH), jnp.int32),
            pltpu.VMEM((_GCH, D), jnp.float32),
            pltpu.SemaphoreType.DMA,
        ],
    )
    def k(x_hbm, perm_hbm, out_hbm, idx_v, rows_v, sem0):
        wid = lax.axis_index("s") * 2 + lax.axis_index("c")
        base = wid * bpw
        pltpu.sync_copy(perm_hbm.at[wid], idx_v)

        def body(g, carry):
            pltpu.async_copy(x_hbm.at[idx_v.at[g]], rows_v, sem0).wait()
            pltpu.sync_copy(rows_v, out_hbm.at[pl.ds(base + g * _GCH, _GCH)])
            return carry

        lax.fori_loop(0, nit, body, 0)

    return k(x, perm.reshape(_NW, nit, _GCH))


def _sc_combine(out_sh, yg, pos0, pos1):
    tpw = T // _NW
    nit = tpw // _CCH

    @functools.partial(
        pl.kernel, mesh=_MESH,
        out_type=jax.ShapeDtypeStruct((T, D), jnp.float32),
        scratch_types=[
            pltpu.VMEM((nit, _CCH), jnp.int32),
            pltpu.VMEM((nit, _CCH), jnp.int32),
            pltpu.VMEM((_CCH, D), jnp.float32),
            pltpu.VMEM((_CCH, D), jnp.float32),
            pltpu.VMEM((_CCH, D), jnp.float32),
            pltpu.SemaphoreType.DMA,
            pltpu.SemaphoreType.DMA,
        ],
    )
    def k(sh_hbm, yg_hbm, p0_hbm, p1_hbm, out_hbm, p0_v, p1_v,
          a_v, b_v, c_v, sem0, sem1):
        wid = lax.axis_index("s") * 2 + lax.axis_index("c")
        base = wid * tpw
        pltpu.sync_copy(p0_hbm.at[wid], p0_v)
        pltpu.sync_copy(p1_hbm.at[wid], p1_v)

        def body(g, carry):
            cpa = pltpu.async_copy(yg_hbm.at[p0_v.at[g]], a_v, sem0)
            cpb = pltpu.async_copy(yg_hbm.at[p1_v.at[g]], b_v, sem1)
            pltpu.sync_copy(sh_hbm.at[pl.ds(base + g * _CCH, _CCH)], c_v)
            cpa.wait()
            cpb.wait()

            def row(r, carry2):
                def col(kk, carry3):
                    for q in range(4):
                        sl = pl.ds(kk * 64 + q * 16, 16)
                        plsc.addupdate(c_v.at[r, sl], a_v[r, sl] + b_v[r, sl])
                    return carry3
                lax.fori_loop(0, D // 64, col, 0)
                return carry2

            lax.fori_loop(0, _CCH, row, 0)
            pltpu.sync_copy(c_v, out_hbm.at[pl.ds(base + g * _CCH, _CCH)])
            return carry

        lax.fori_loop(0, nit, body, 0)

    return k(out_sh, yg, pos0.reshape(_NW, nit, _CCH), pos1.reshape(_NW, nit, _CCH))


def _routed_body(be_ref, x_ref, wg_ref, wu_ref, wd_ref, ws_ref, out_ref):
    b = pl.program_id(0)
    f = pl.program_id(1)

    @pl.when(f == 0)
    def _():
        out_ref[...] = jnp.zeros_like(out_ref)

    @pl.when(be_ref[b] < E)
    def _():
        x = x_ref[...]
        g = jnp.dot(x, wg_ref[0], preferred_element_type=jnp.float32)
        u = jnp.dot(x, wu_ref[0], preferred_element_type=jnp.float32)
        h = (g * jax.nn.sigmoid(g)) * u
        out_ref[...] += jnp.dot(h, wd_ref[0], preferred_element_type=jnp.float32)

    @pl.when(f == NF - 1)
    def _():
        out_ref[...] *= ws_ref[0, 0, :][:, None]


def _routed_mlp(block_expert, xg, Wg, Wu, Wd, wslot):
    grid_spec = pltpu.PrefetchScalarGridSpec(
        num_scalar_prefetch=1,
        grid=(NB, NF),
        in_specs=[
            pl.BlockSpec((BM, D), lambda b, f, be: (b, 0)),
            pl.BlockSpec((1, D, BF), lambda b, f, be: (jnp.minimum(be[b], E - 1), 0, f)),
            pl.BlockSpec((1, D, BF), lambda b, f, be: (jnp.minimum(be[b], E - 1), 0, f)),
            pl.BlockSpec((1, BF, D), lambda b, f, be: (jnp.minimum(be[b], E - 1), f, 0)),
            pl.BlockSpec((1, 1, BM), lambda b, f, be: (b, 0, 0)),
        ],
        out_specs=pl.BlockSpec((BM, D), lambda b, f, be: (b, 0)),
    )
    return pl.pallas_call(
        _routed_body,
        grid_spec=grid_spec,
        out_shape=jax.ShapeDtypeStruct((P, D), jnp.float32),
    )(block_expert, xg, Wg, Wu, Wd, wslot.reshape(NB, 1, BM))


def _shared_body(x_ref, wg_ref, wu_ref, wd_ref, out_ref):
    si = pl.program_id(1)
    f = pl.program_id(2)

    @pl.when((si == 0) & (f == 0))
    def _():
        out_ref[...] = jnp.zeros_like(out_ref)

    x = x_ref[...]
    g = jnp.dot(x, wg_ref[0], preferred_element_type=jnp.float32)
    u = jnp.dot(x, wu_ref[0], preferred_element_type=jnp.float32)
    h = (g * jax.nn.sigmoid(g)) * u
    out_ref[...] += jnp.dot(h, wd_ref[0], preferred_element_type=jnp.float32)


def _shared_mlp(x, Wg_s, Wu_s, Wd_s):
    return pl.pallas_call(
        _shared_body,
        grid=(T // BM2, NSH, NF),
        in_specs=[
            pl.BlockSpec((BM2, D), lambda tb, si, f: (tb, 0)),
            pl.BlockSpec((1, D, BF), lambda tb, si, f: (si, 0, f)),
            pl.BlockSpec((1, D, BF), lambda tb, si, f: (si, 0, f)),
            pl.BlockSpec((1, BF, D), lambda tb, si, f: (si, f, 0)),
        ],
        out_specs=pl.BlockSpec((BM2, D), lambda tb, si, f: (tb, 0)),
        out_shape=jax.ShapeDtypeStruct((T, D), jnp.float32),
    )(x, Wg_s, Wu_s, Wd_s)


def kernel(hidden_states, gate_w, Wg, Wu, Wd, Wg_s, Wu_s, Wd_s):
    b, s, d = hidden_states.shape
    x = hidden_states.reshape(-1, d)

    logits = x @ gate_w.T
    scores = jax.nn.sigmoid(logits)
    topk_w, topk_idx = jax.lax.top_k(scores, K)
    topk_w = topk_w / jnp.sum(topk_w, axis=-1, keepdims=True)

    flat_e = topk_idx.reshape(-1).astype(jnp.int32)
    flat_w = topk_w.reshape(-1)
    oh = (flat_e[:, None] == jnp.arange(E, dtype=jnp.int32)[None, :]).astype(jnp.int32)
    csum = jnp.cumsum(oh, axis=0)
    counts = csum[-1]
    padded = ((counts + BM - 1) // BM) * BM
    cum_padded = jnp.cumsum(padded)
    p_off = cum_padded - padded
    rank = jnp.take_along_axis(csum, flat_e[:, None], 1)[:, 0] - 1
    dest = p_off[flat_e] + rank
    tok = jnp.arange(T * K, dtype=jnp.int32) // K
    perm = (jnp.arange(P, dtype=jnp.int32) % T).at[dest].set(tok)
    wslot = jnp.zeros((P,), jnp.float32).at[dest].set(flat_w)
    pos = dest.reshape(T, K)
    block_expert = jnp.searchsorted(
        cum_padded, jnp.arange(NB, dtype=jnp.int32) * BM, side="right"
    ).astype(jnp.int32)

    xg = _sc_gather(x, perm)
    yg = _routed_mlp(block_expert, xg, Wg, Wu, Wd, wslot)
    out_sh = _shared_mlp(x, Wg_s, Wu_s, Wd_s)
    out = _sc_combine(out_sh, yg, pos[:, 0], pos[:, 1])
    return out.reshape(b, s, d)


# speedup vs baseline: 1.4405x; 1.0338x over previous
import functools

import jax
import jax.numpy as jnp
from jax import lax
from jax.experimental import pallas as pl
from jax.experimental.pallas import tpu as pltpu
from jax.experimental.pallas import tpu_sc as plsc

D, F, E, K, NSH = 2048, 5120, 8, 2, 2
T = 2048
BM = 256
BF = 512
NF = F // BF
NB = 23
P = NB * BM
BM2 = 512

_NW = 32
_GCH = 8
_CCH = 16

_MESH = plsc.VectorSubcoreMesh(core_axis_name="c", subcore_axis_name="s", num_cores=2, num_subcores=16)


def _sc_metadata(eflat, wflat):
    NCH = (T * K) // 16

    @functools.partial(
        pl.kernel, mesh=_MESH,
        compiler_params=pltpu.CompilerParams(needs_layout_passes=False),
        out_type=(
            jax.ShapeDtypeStruct((P,), jnp.int32),
            jax.ShapeDtypeStruct((P,), jnp.float32),
            jax.ShapeDtypeStruct((T,), jnp.int32),
            jax.ShapeDtypeStruct((T,), jnp.int32),
            jax.ShapeDtypeStruct((32,), jnp.int32),
        ),
        scratch_types=[
            pltpu.VMEM((T * K,), jnp.int32),
            pltpu.VMEM((T * K,), jnp.float32),
            pltpu.VMEM((P,), jnp.int32),
            pltpu.VMEM((P,), jnp.float32),
            pltpu.VMEM((T,), jnp.int32),
            pltpu.VMEM((T,), jnp.int32),
            pltpu.VMEM((16,), jnp.int32),
            pltpu.VMEM((32,), jnp.int32),
        ],
    )
    def k(e_hbm, w_hbm, perm_hbm, ws_hbm, p0_hbm, p1_hbm, be_hbm,
          e_v, w_v, perm_v, ws_v, pos0_v, pos1_v, base_ref, be_v):
        wid = lax.axis_index("s") * 2 + lax.axis_index("c")

        @pl.when(wid == 0)
        def _():
            pltpu.sync_copy(e_hbm, e_v)
            pltpu.sync_copy(w_hbm, w_v)
            lanes = lax.iota(jnp.int32, 16)

            def initp(g, carry):
                perm_v[pl.ds(g * 16, 16)] = (g * 16 + lanes) & (T - 1)
                ws_v[pl.ds(g * 16, 16)] = jnp.zeros((16,), jnp.float32)
                return carry

            lax.fori_loop(0, P // 16, initp, 0)

            def hist_body(g, hist):
                ev = e_v[pl.ds(g * 16, 16)]
                for e in range(E):
                    c = plsc.all_reduce_population_count(ev == e)
                    hist = hist + jnp.where(lanes == e, c, 0)
                return hist

            hist = lax.fori_loop(0, NCH, hist_body, jnp.zeros((16,), jnp.int32))
            padded = ((hist + (BM - 1)) >> 8) << 8
            cum = plsc.cumsum(padded)
            poff = cum - padded
            base_ref[...] = poff

            for ch in range(2):
                bstart = (ch * 16 + lanes) * BM
                acc = jnp.zeros((16,), jnp.int32)
                for e in range(E):
                    acc = acc + jnp.where(bstart >= cum[e], 1, 0)
                be_v[pl.ds(ch * 16, 16)] = acc
            pltpu.sync_copy(be_v, be_hbm)

            def body(g, carry):
                ev = e_v[pl.ds(g * 16, 16)]
                wv = w_v[pl.ds(g * 16, 16)]
                basev = plsc.load_gather(base_ref, [ev])
                rank = jnp.zeros((16,), jnp.int32)
                newbase = base_ref[...]
                for e in range(E):
                    m = ev == e
                    cs = plsc.cumsum(jnp.where(m, 1, 0))
                    rank = jnp.where(m, cs - 1, rank)
                    cnt = plsc.all_reduce_population_count(m)
                    newbase = newbase + jnp.where(lanes == e, cnt, 0)
                base_ref[...] = newbase
                dest = basev + rank
                tokv = (g * 16 + lanes) >> 1
                plsc.store_scatter(perm_v, [dest], tokv)
                plsc.store_scatter(ws_v, [dest], wv)
                even = (lanes & 1) == 0
                plsc.store_scatter(pos0_v, [tokv], dest, mask=even)
                plsc.store_scatter(pos1_v, [tokv], dest, mask=jnp.logical_not(even))
                return carry

            lax.fori_loop(0, NCH, body, 0)
            pltpu.sync_copy(perm_v, perm_hbm)
            pltpu.sync_copy(ws_v, ws_hbm)
            pltpu.sync_copy(pos0_v, p0_hbm)
            pltpu.sync_copy(pos1_v, p1_hbm)

    return k(eflat, wflat)


def _sc_gather(x, perm):
    bpw = P // _NW
    nit = bpw // _GCH

    @functools.partial(
        pl.kernel, mesh=_MESH,
        out_type=jax.ShapeDtypeStruct((P, D), jnp.float32),
        scratch_types=[
            pltpu.VMEM((nit, _GCH), jnp.int32),
            pltpu.VMEM((_GCH, D), jnp.float32),
            pltpu.SemaphoreType.DMA,
        ],
    )
    def k(x_hbm, perm_hbm, out_hbm, idx_v, rows_v, sem0):
        wid = lax.axis_index("s") * 2 + lax.axis_index("c")
        base = wid * bpw
        pltpu.sync_copy(perm_hbm.at[wid], idx_v)

        def body(g, carry):
            pltpu.async_copy(x_hbm.at[idx_v.at[g]], rows_v, sem0).wait()
            pltpu.sync_copy(rows_v, out_hbm.at[pl.ds(base + g * _GCH, _GCH)])
            return carry

        lax.fori_loop(0, nit, body, 0)

    return k(x, perm.reshape(_NW, nit, _GCH))


def _sc_combine(out_sh, yg, pos0, pos1):
    tpw = T // _NW
    nit = tpw // _CCH

    @functools.partial(
        pl.kernel, mesh=_MESH,
        out_type=jax.ShapeDtypeStruct((T, D), jnp.float32),
        scratch_types=[
            pltpu.VMEM((nit, _CCH), jnp.int32),
            pltpu.VMEM((nit, _CCH), jnp.int32),
            pltpu.VMEM((_CCH, D), jnp.float32),
            pltpu.VMEM((_CCH, D), jnp.float32),
            pltpu.VMEM((_CCH, D), jnp.float32),
            pltpu.SemaphoreType.DMA,
            pltpu.SemaphoreType.DMA,
        ],
    )
    def k(sh_hbm, yg_hbm, p0_hbm, p1_hbm, out_hbm, p0_v, p1_v,
          a_v, b_v, c_v, sem0, sem1):
        wid = lax.axis_index("s") * 2 + lax.axis_index("c")
        base = wid * tpw
        pltpu.sync_copy(p0_hbm.at[wid], p0_v)
        pltpu.sync_copy(p1_hbm.at[wid], p1_v)

        def body(g, carry):
            cpa = pltpu.async_copy(yg_hbm.at[p0_v.at[g]], a_v, sem0)
            cpb = pltpu.async_copy(yg_hbm.at[p1_v.at[g]], b_v, sem1)
            pltpu.sync_copy(sh_hbm.at[pl.ds(base + g * _CCH, _CCH)], c_v)
            cpa.wait()
            cpb.wait()

            def row(r, carry2):
                def col(kk, carry3):
                    for q in range(4):
                        sl = pl.ds(kk * 64 + q * 16, 16)
                        plsc.addupdate(c_v.at[r, sl], a_v[r, sl] + b_v[r, sl])
                    return carry3
                lax.fori_loop(0, D // 64, col, 0)
                return carry2

            lax.fori_loop(0, _CCH, row, 0)
            pltpu.sync_copy(c_v, out_hbm.at[pl.ds(base + g * _CCH, _CCH)])
            return carry

        lax.fori_loop(0, nit, body, 0)

    return k(out_sh, yg, pos0.reshape(_NW, nit, _CCH), pos1.reshape(_NW, nit, _CCH))


def _routed_body(be_ref, x_ref, wg_ref, wu_ref, wd_ref, ws_ref, out_ref):
    b = pl.program_id(0)
    f = pl.program_id(1)

    @pl.when(f == 0)
    def _():
        out_ref[...] = jnp.zeros_like(out_ref)

    @pl.when(be_ref[b] < E)
    def _():
        x = x_ref[...]
        g = jnp.dot(x, wg_ref[0], preferred_element_type=jnp.float32)
        u = jnp.dot(x, wu_ref[0], preferred_element_type=jnp.float32)
        h = (g * jax.nn.sigmoid(g)) * u
        out_ref[...] += jnp.dot(h, wd_ref[0], preferred_element_type=jnp.float32)

    @pl.when(f == NF - 1)
    def _():
        out_ref[...] *= ws_ref[0, 0, :][:, None]


def _routed_mlp(block_expert, xg, Wg, Wu, Wd, wslot):
    grid_spec = pltpu.PrefetchScalarGridSpec(
        num_scalar_prefetch=1,
        grid=(NB, NF),
        in_specs=[
            pl.BlockSpec((BM, D), lambda b, f, be: (b, 0)),
            pl.BlockSpec((1, D, BF), lambda b, f, be: (jnp.minimum(be[b], E - 1), 0, f)),
            pl.BlockSpec((1, D, BF), lambda b, f, be: (jnp.minimum(be[b], E - 1), 0, f)),
            pl.BlockSpec((1, BF, D), lambda b, f, be: (jnp.minimum(be[b], E - 1), f, 0)),
            pl.BlockSpec((1, 1, BM), lambda b, f, be: (b, 0, 0)),
        ],
        out_specs=pl.BlockSpec((BM, D), lambda b, f, be: (b, 0)),
    )
    return pl.pallas_call(
        _routed_body,
        grid_spec=grid_spec,
        out_shape=jax.ShapeDtypeStruct((P, D), jnp.float32),
    )(block_expert, xg, Wg, Wu, Wd, wslot.reshape(NB, 1, BM))


def _shared_body(x_ref, wg_ref, wu_ref, wd_ref, out_ref):
    si = pl.program_id(1)
    f = pl.program_id(2)

    @pl.when((si == 0) & (f == 0))
    def _():
        out_ref[...] = jnp.zeros_like(out_ref)

    x = x_ref[...]
    g = jnp.dot(x, wg_ref[0], preferred_element_type=jnp.float32)
    u = jnp.dot(x, wu_ref[0], preferred_element_type=jnp.float32)
    h = (g * jax.nn.sigmoid(g)) * u
    out_ref[...] += jnp.dot(h, wd_ref[0], preferred_element_type=jnp.float32)


def _shared_mlp(x, Wg_s, Wu_s, Wd_s):
    return pl.pallas_call(
        _shared_body,
        grid=(T // BM2, NSH, NF),
        in_specs=[
            pl.BlockSpec((BM2, D), lambda tb, si, f: (tb, 0)),
            pl.BlockSpec((1, D, BF), lambda tb, si, f: (si, 0, f)),
            pl.BlockSpec((1, D, BF), lambda tb, si, f: (si, 0, f)),
            pl.BlockSpec((1, BF, D), lambda tb, si, f: (si, f, 0)),
        ],
        out_specs=pl.BlockSpec((BM2, D), lambda tb, si, f: (tb, 0)),
        out_shape=jax.ShapeDtypeStruct((T, D), jnp.float32),
    )(x, Wg_s, Wu_s, Wd_s)


def kernel(hidden_states, gate_w, Wg, Wu, Wd, Wg_s, Wu_s, Wd_s):
    b, s, d = hidden_states.shape
    x = hidden_states.reshape(-1, d)

    logits = x @ gate_w.T
    scores = jax.nn.sigmoid(logits)
    topk_w, topk_idx = jax.lax.top_k(scores, K)
    topk_w = topk_w / jnp.sum(topk_w, axis=-1, keepdims=True)

    perm, wslot, pos0, pos1, block_expert = _sc_metadata(
        topk_idx.reshape(-1).astype(jnp.int32), topk_w.reshape(-1))

    xg = _sc_gather(x, perm)
    yg = _routed_mlp(block_expert, xg, Wg, Wu, Wd, wslot)
    out_sh = _shared_mlp(x, Wg_s, Wu_s, Wd_s)
    out = _sc_combine(out_sh, yg, pos0, pos1)
    return out.reshape(b, s, d)


# speedup vs baseline: 1.4433x; 1.0019x over previous
import functools

import jax
import jax.numpy as jnp
from jax import lax
from jax.experimental import pallas as pl
from jax.experimental.pallas import tpu as pltpu
from jax.experimental.pallas import tpu_sc as plsc

D, F, E, K, NSH = 2048, 5120, 8, 2, 2
T = 2048
BM = 256
BF = 512
NF = F // BF
NB = 23
P = NB * BM
BM2 = 512

_NW = 32
_GCH = 8
_CCH = 16

_MESH = plsc.VectorSubcoreMesh(core_axis_name="c", subcore_axis_name="s", num_cores=2, num_subcores=16)


def _sc_dispatch_gather(x, eflat, wflat):
    NCH = (T * K) // 16
    bpw = P // _NW
    nit = bpw // _GCH

    @functools.partial(
        pl.kernel, mesh=_MESH,
        compiler_params=pltpu.CompilerParams(needs_layout_passes=False),
        out_type=(
            jax.ShapeDtypeStruct((P, D), jnp.float32),
            jax.ShapeDtypeStruct((P,), jnp.float32),
            jax.ShapeDtypeStruct((T,), jnp.int32),
            jax.ShapeDtypeStruct((T,), jnp.int32),
            jax.ShapeDtypeStruct((32,), jnp.int32),
        ),
        scratch_types=[
            pltpu.VMEM((T * K,), jnp.int32),
            pltpu.VMEM((T * K,), jnp.float32),
            pltpu.VMEM((P,), jnp.int32),
            pltpu.VMEM((P,), jnp.float32),
            pltpu.VMEM((T,), jnp.int32),
            pltpu.VMEM((T,), jnp.int32),
            pltpu.VMEM((16,), jnp.int32),
            pltpu.VMEM((32,), jnp.int32),
            pltpu.VMEM((2, _GCH, D), jnp.float32),
            pltpu.SemaphoreType.DMA,
            pltpu.SemaphoreType.DMA,
        ],
    )
    def k(x_hbm, e_hbm, w_hbm, xg_hbm, ws_hbm, p0_hbm, p1_hbm, be_hbm,
          e_v, w_v, perm_v, ws_v, pos0_v, pos1_v, base_ref, be_v, rows_v,
          sem0, sem1):
        wid = lax.axis_index("s") * 2 + lax.axis_index("c")
        base = wid * bpw
        pltpu.sync_copy(e_hbm, e_v)
        pltpu.sync_copy(w_hbm, w_v)
        lanes = lax.iota(jnp.int32, 16)

        def initp(g, carry):
            perm_v[pl.ds(g * 16, 16)] = (g * 16 + lanes) & (T - 1)
            ws_v[pl.ds(g * 16, 16)] = jnp.zeros((16,), jnp.float32)
            return carry

        lax.fori_loop(0, P // 16, initp, 0)

        def hist_body(g, hist):
            ev = e_v[pl.ds(g * 16, 16)]
            for e in range(E):
                c = plsc.all_reduce_population_count(ev == e)
                hist = hist + jnp.where(lanes == e, c, 0)
            return hist

        hist = lax.fori_loop(0, NCH, hist_body, jnp.zeros((16,), jnp.int32))
        padded = ((hist + (BM - 1)) >> 8) << 8
        cum = plsc.cumsum(padded)
        poff = cum - padded
        base_ref[...] = poff

        @pl.when(wid == 0)
        def _():
            for ch in range(2):
                bstart = (ch * 16 + lanes) * BM
                acc = jnp.zeros((16,), jnp.int32)
                for e in range(E):
                    acc = acc + jnp.where(bstart >= cum[e], 1, 0)
                be_v[pl.ds(ch * 16, 16)] = acc
            pltpu.sync_copy(be_v, be_hbm)

        def body(g, carry):
            ev = e_v[pl.ds(g * 16, 16)]
            wv = w_v[pl.ds(g * 16, 16)]
            basev = plsc.load_gather(base_ref, [ev])
            rank = jnp.zeros((16,), jnp.int32)
            newbase = base_ref[...]
            for e in range(E):
                m = ev == e
                cs = plsc.cumsum(jnp.where(m, 1, 0))
                rank = jnp.where(m, cs - 1, rank)
                cnt = plsc.all_reduce_population_count(m)
                newbase = newbase + jnp.where(lanes == e, cnt, 0)
            base_ref[...] = newbase
            dest = basev + rank
            tokv = (g * 16 + lanes) >> 1
            plsc.store_scatter(perm_v, [dest], tokv)
            plsc.store_scatter(ws_v, [dest], wv)
            even = (lanes & 1) == 0
            plsc.store_scatter(pos0_v, [tokv], dest, mask=even)
            plsc.store_scatter(pos1_v, [tokv], dest, mask=jnp.logical_not(even))
            return carry

        lax.fori_loop(0, NCH, body, 0)

        @pl.when(wid == 0)
        def _():
            pltpu.sync_copy(ws_v, ws_hbm)
            pltpu.sync_copy(pos0_v, p0_hbm)
            pltpu.sync_copy(pos1_v, p1_hbm)

        def idx(g):
            return perm_v.at[pl.ds(base + g * _GCH, _GCH)]

        pltpu.async_copy(x_hbm.at[idx(0)], rows_v.at[0], sem0)

        def gbody(h, carry):
            g0 = 2 * h

            @pl.when(g0 + 1 < nit)
            def _():
                pltpu.async_copy(x_hbm.at[idx(g0 + 1)], rows_v.at[1], sem1)

            pltpu.make_async_copy(x_hbm.at[idx(0)], rows_v.at[0], sem0).wait()
            pltpu.sync_copy(rows_v.at[0], xg_hbm.at[pl.ds(base + g0 * _GCH, _GCH)])

            @pl.when(g0 + 2 < nit)
            def _():
                pltpu.async_copy(x_hbm.at[idx(g0 + 2)], rows_v.at[0], sem0)

            @pl.when(g0 + 1 < nit)
            def _():
                pltpu.make_async_copy(x_hbm.at[idx(0)], rows_v.at[1], sem1).wait()
                pltpu.sync_copy(rows_v.at[1],
                                xg_hbm.at[pl.ds(base + (g0 + 1) * _GCH, _GCH)])

            return carry

        lax.fori_loop(0, (nit + 1) // 2, gbody, 0)

    return k(x, eflat, wflat)


def _sc_combine(out_sh, yg, pos0, pos1):
    tpw = T // _NW
    nit = tpw // _CCH

    @functools.partial(
        pl.kernel, mesh=_MESH,
        out_type=jax.ShapeDtypeStruct((T, D), jnp.float32),
        scratch_types=[
            pltpu.VMEM((nit, _CCH), jnp.int32),
            pltpu.VMEM((nit, _CCH), jnp.int32),
            pltpu.VMEM((_CCH, D), jnp.float32),
            pltpu.VMEM((_CCH, D), jnp.float32),
            pltpu.VMEM((_CCH, D), jnp.float32),
            pltpu.SemaphoreType.DMA,
            pltpu.SemaphoreType.DMA,
        ],
    )
    def k(sh_hbm, yg_hbm, p0_hbm, p1_hbm, out_hbm, p0_v, p1_v,
          a_v, b_v, c_v, sem0, sem1):
        wid = lax.axis_index("s") * 2 + lax.axis_index("c")
        base = wid * tpw
        pltpu.sync_copy(p0_hbm.at[wid], p0_v)
        pltpu.sync_copy(p1_hbm.at[wid], p1_v)

        def body(g, carry):
            cpa = pltpu.async_copy(yg_hbm.at[p0_v.at[g]], a_v, sem0)
            cpb = pltpu.async_copy(yg_hbm.at[p1_v.at[g]], b_v, sem1)
            pltpu.sync_copy(sh_hbm.at[pl.ds(base + g * _CCH, _CCH)], c_v)
            cpa.wait()
            cpb.wait()

            def row(r, carry2):
                def col(kk, carry3):
                    for q in range(4):
                        sl = pl.ds(kk * 64 + q * 16, 16)
                        plsc.addupdate(c_v.at[r, sl], a_v[r, sl] + b_v[r, sl])
                    return carry3
                lax.fori_loop(0, D // 64, col, 0)
                return carry2

            lax.fori_loop(0, _CCH, row, 0)
            pltpu.sync_copy(c_v, out_hbm.at[pl.ds(base + g * _CCH, _CCH)])
            return carry

        lax.fori_loop(0, nit, body, 0)

    return k(out_sh, yg, pos0.reshape(_NW, nit, _CCH), pos1.reshape(_NW, nit, _CCH))


def _routed_body(be_ref, x_ref, wg_ref, wu_ref, wd_ref, ws_ref, out_ref):
    b = pl.program_id(0)
    f = pl.program_id(1)

    @pl.when(f == 0)
    def _():
        out_ref[...] = jnp.zeros_like(out_ref)

    @pl.when(be_ref[b] < E)
    def _():
        x = x_ref[...]
        g = jnp.dot(x, wg_ref[0], preferred_element_type=jnp.float32)
        u = jnp.dot(x, wu_ref[0], preferred_element_type=jnp.float32)
        h = (g * jax.nn.sigmoid(g)) * u
        out_ref[...] += jnp.dot(h, wd_ref[0], preferred_element_type=jnp.float32)

    @pl.when(f == NF - 1)
    def _():
        out_ref[...] *= ws_ref[0, 0, :][:, None]


def _routed_mlp(block_expert, xg, Wg, Wu, Wd, wslot):
    grid_spec = pltpu.PrefetchScalarGridSpec(
        num_scalar_prefetch=1,
        grid=(NB, NF),
        in_specs=[
            pl.BlockSpec((BM, D), lambda b, f, be: (b, 0)),
            pl.BlockSpec((1, D, BF), lambda b, f, be: (jnp.minimum(be[b], E - 1), 0, f)),
            pl.BlockSpec((1, D, BF), lambda b, f, be: (jnp.minimum(be[b], E - 1), 0, f)),
            pl.BlockSpec((1, BF, D), lambda b, f, be: (jnp.minimum(be[b], E - 1), f, 0)),
            pl.BlockSpec((1, 1, BM), lambda b, f, be: (b, 0, 0)),
        ],
        out_specs=pl.BlockSpec((BM, D), lambda b, f, be: (b, 0)),
    )
    return pl.pallas_call(
        _routed_body,
        grid_spec=grid_spec,
        out_shape=jax.ShapeDtypeStruct((P, D), jnp.float32),
    )(block_expert, xg, Wg, Wu, Wd, wslot.reshape(NB, 1, BM))


def _shared_body(x_ref, wg_ref, wu_ref, wd_ref, out_ref):
    si = pl.program_id(1)
    f = pl.program_id(2)

    @pl.when((si == 0) & (f == 0))
    def _():
        out_ref[...] = jnp.zeros_like(out_ref)

    x = x_ref[...]
    g = jnp.dot(x, wg_ref[0], preferred_element_type=jnp.float32)
    u = jnp.dot(x, wu_ref[0], preferred_element_type=jnp.float32)
    h = (g * jax.nn.sigmoid(g)) * u
    out_ref[...] += jnp.dot(h, wd_ref[0], preferred_element_type=jnp.float32)


def _shared_mlp(x, Wg_s, Wu_s, Wd_s):
    return pl.pallas_call(
        _shared_body,
        grid=(T // BM2, NSH, NF),
        in_specs=[
            pl.BlockSpec((BM2, D), lambda tb, si, f: (tb, 0)),
            pl.BlockSpec((1, D, BF), lambda tb, si, f: (si, 0, f)),
            pl.BlockSpec((1, D, BF), lambda tb, si, f: (si, 0, f)),
            pl.BlockSpec((1, BF, D), lambda tb, si, f: (si, f, 0)),
        ],
        out_specs=pl.BlockSpec((BM2, D), lambda tb, si, f: (tb, 0)),
        out_shape=jax.ShapeDtypeStruct((T, D), jnp.float32),
    )(x, Wg_s, Wu_s, Wd_s)


def kernel(hidden_states, gate_w, Wg, Wu, Wd, Wg_s, Wu_s, Wd_s):
    b, s, d = hidden_states.shape
    x = hidden_states.reshape(-1, d)

    logits = x @ gate_w.T
    scores = jax.nn.sigmoid(logits)
    topk_w, topk_idx = jax.lax.top_k(scores, K)
    topk_w = topk_w / jnp.sum(topk_w, axis=-1, keepdims=True)

    xg, wslot, pos0, pos1, block_expert = _sc_dispatch_gather(
        x, topk_idx.reshape(-1).astype(jnp.int32), topk_w.reshape(-1))

    yg = _routed_mlp(block_expert, xg, Wg, Wu, Wd, wslot)
    out_sh = _shared_mlp(x, Wg_s, Wu_s, Wd_s)
    out = _sc_combine(out_sh, yg, pos0, pos1)
    return out.reshape(b, s, d)
